# baseline (speedup 1.0000x reference)
import functools
import itertools

import numpy as np
import jax
import jax.numpy as jnp
from jax.experimental import pallas as pl
from jax.experimental.pallas import tpu as pltpu

D = 1024
DFF = 2048
_HEXG_NP = np.array(list(itertools.product([-1.0, 1.0], repeat=6)),
                    dtype=np.float32)
_GIDX = [[63, 62], [19, 21], [0, 8]]
_ANCH_NP = np.stack([_HEXG_NP[np.array(i)].mean(0) for i in _GIDX])
_ANCH_NP = (_ANCH_NP /
            np.linalg.norm(_ANCH_NP, axis=-1, keepdims=True)).astype(np.float32)

_BN = 512
_BA = 1024
_BB = 1024
_BQ = 1024

_F32 = jnp.float32
_BF16 = jnp.bfloat16
_HI = jax.lax.Precision.HIGHEST


def _std(x):
    m = jnp.mean(x, axis=-1, keepdims=True)
    xc = x - m
    v = jnp.mean(xc * xc, axis=-1, keepdims=True)
    return xc * jax.lax.rsqrt(v + 1e-5)


def _dott(a, w):
    return jax.lax.dot_general(a, w, (((1,), (1,)), ((), ())),
                               preferred_element_type=_F32)


def _dotb(a, w):
    return jax.lax.dot_general(a, w, (((1,), (1,)), ((), ())),
                               preferred_element_type=_BF16)


def _place(val, k, width):
    lane = jax.lax.broadcasted_iota(jnp.int32, (1, width), 1)
    return jnp.where(lane == k, val, 0.0)


def _router_kernel(x_ref, wsm_ref, bsm_ref, sc_ref, hexg_ref, anch_ref,
                   xhat_ref, cw_ref, ls_ref):
    i = pl.program_id(0)
    x = x_ref[...]
    xh = _std(x)
    xhat_ref[...] = xh.astype(_BF16)
    logits = jax.lax.dot_general(
        xh, wsm_ref[...], (((1,), (1,)), ((), ())),
        preferred_element_type=_F32, precision=_HI) + bsm_ref[...]
    hexg = hexg_ref[...]
    anch = anch_ref[...]
    sb = jnp.tanh(logits[:, 0:6])
    temp = sc_ref[0:1, 0:1]
    sim = jax.lax.dot_general(sb, hexg, (((1,), (1,)), ((), ())),
                              preferred_element_type=_F32,
                              precision=_HI) / temp
    sim = sim - jnp.max(sim, axis=1, keepdims=True)
    se = jnp.exp(sim)
    hw = se / jnp.sum(se, axis=1, keepdims=True)
    shex = jnp.dot(hw, hexg, preferred_element_type=_F32,
                   precision=_HI)
    gs = jax.lax.dot_general(shex, anch, (((1,), (1,)), ((), ())),
                             preferred_element_type=_F32,
                             precision=_HI)
    gs = gs - jnp.max(gs, axis=1, keepdims=True)
    ge = jnp.exp(gs)
    gw = ge / jnp.sum(ge, axis=1, keepdims=True)

    row = (_place(jnp.sum(gw[:, 0:1]), 0, 128)
           + _place(jnp.sum(gw[:, 1:2]), 1, 128)
           + _place(jnp.sum(gw[:, 2:3]), 2, 128))
    for g in range(3):
        pair = logits[:, 6 + 2 * g:8 + 2 * g]
        mx = jnp.max(pair, axis=1, keepdims=True)
        pe = jnp.exp(pair - mx)
        ps = jnp.sum(pe, axis=1, keepdims=True)
        w = pe / ps
        sp = w / (jnp.sum(w, axis=1, keepdims=True) + 1e-8)
        row = (row + _place(jnp.sum(w[:, 0:1]), 3 + 2 * g, 128)
               + _place(jnp.sum(w[:, 1:2]), 4 + 2 * g, 128))
        for e in range(2):
            j = 2 * g + e
            cm = gw[:, g:g + 1] * sp[:, e:e + 1]
            cj = _place(cm, 0, 8)
            if g == 0:
                cj = cj + _place(sp[:, e:e + 1], 1, 8)
            if g == 2:
                cj = cj + _place(sp[:, e:e + 1], 2, 8)
            cw_ref[j] = cj

    @pl.when(i == 0)
    def _():
        ls_ref[...] = jnp.zeros_like(ls_ref)

    ls_ref[...] += row


def _expert_kernel(xh_ref, wg_ref, wv_ref, wo_ref, bg_ref, bv_ref, cw_ref,
                   moe_ref, xab_ref):
    j = pl.program_id(1)
    xh = xh_ref[...]
    gate = _dott(xh, wg_ref[0]) + bg_ref[0]
    val = _dott(xh, wv_ref[0]) + bv_ref[0]
    h = (gate * jax.nn.sigmoid(gate) * val).astype(_BF16)
    eo = _dott(h, wo_ref[0])
    c = cw_ref[0]

    @pl.when(j == 0)
    def _():
        moe_ref[...] = (c[:, 0:1] * eo).astype(_BF16)
        xab_ref[0] = (c[:, 1:2] * eo).astype(_BF16)

    @pl.when(j > 0)
    def _():
        moe_ref[...] = (moe_ref[...] + c[:, 0:1] * eo).astype(_BF16)

    @pl.when(j == 1)
    def _():
        xab_ref[0] = (xab_ref[0] + c[:, 1:2] * eo).astype(_BF16)

    @pl.when(j == 4)
    def _():
        xab_ref[1] = (c[:, 2:3] * eo).astype(_BF16)

    @pl.when(j == 5)
    def _():
        xab_ref[1] = (xab_ref[1] + c[:, 2:3] * eo).astype(_BF16)


def _qkv_kernel(xab_ref, wqf_ref, wkf_ref, wvf_ref, wqb_ref, wkb_ref,
                wvb_ref, bias_ref, q_ref, k_ref, v_ref):
    na = _std(xab_ref[0]).astype(_BF16)
    nb_ = _std(xab_ref[1]).astype(_BF16)
    q_ref[0] = (_dott(na, wqf_ref[...]) + bias_ref[0:1]).astype(_BF16)
    k_ref[0] = (_dott(nb_, wkf_ref[...]) + bias_ref[1:2]).astype(_BF16)
    v_ref[0] = (_dott(nb_, wvf_ref[...]) + bias_ref[2:3]).astype(_BF16)
    q_ref[1] = (_dott(nb_, wqb_ref[...]) + bias_ref[3:4]).astype(_BF16)
    k_ref[1] = (_dott(na, wkb_ref[...]) + bias_ref[4:5]).astype(_BF16)
    v_ref[1] = (_dott(na, wvb_ref[...]) + bias_ref[5:6]).astype(_BF16)


def _attn_kernel(q_ref, k_ref, v_ref, o_ref, *, qi, kw):
    q = q_ref[0, 0]
    k = k_ref[0, 0]
    v = v_ref[0, 0]
    s = jax.lax.dot_general(q, k, (((1,), (1,)), ((), ())),
                            preferred_element_type=_F32) * (1.0 / 16.0)
    qpos = qi * _BQ + jax.lax.broadcasted_iota(jnp.int32, (_BQ, kw), 0)
    kpos = jax.lax.broadcasted_iota(jnp.int32, (_BQ, kw), 1)
    s = jnp.where(kpos > qpos, -1e30, s)
    mx = jnp.max(s, axis=1, keepdims=True)
    p = jnp.exp(s - mx)
    l = jnp.sum(p, axis=1, keepdims=True)
    o = jnp.dot(p.astype(_BF16), v, preferred_element_type=_F32) / l
    o_ref[0, 0] = o.astype(_BF16)


def _final_kernel(x_ref, moe_ref, xab_ref, o_ref, wof_ref, wob_ref,
                  sc_ref, wg_ref, bg_ref, wv_ref, bv_ref, wo2_ref, out_ref):
    alpha = sc_ref[0:1, 1:2]
    gsig = sc_ref[0:1, 2:3]
    fwd = _dott(o_ref[0], wof_ref[...])
    bwd = _dott(o_ref[1], wob_ref[...])
    crossed = (alpha * (xab_ref[0] + fwd)
               + (1.0 - alpha) * (xab_ref[1] + bwd))
    hh = _std(crossed).astype(_BF16)
    g = _dott(hh, wg_ref[...]) + bg_ref[...]
    vv = _dott(hh, wv_ref[...]) + bv_ref[...]
    h = (g * jax.nn.sigmoid(g) * vv).astype(_BF16)
    ffn = _dott(h, wo2_ref[...])
    bout = (crossed + ffn) * gsig
    out_ref[...] = x_ref[...] + moe_ref[...] + 0.5 * bout


def kernel(x, params):
    p = params
    b, t, d = x.shape
    n = b * t
    xf = x.reshape(n, d)

    rows = [p["q6_W"] * p["gr_norm_w"][None, :]]
    bias = [p["q6_W"] @ p["gr_norm_b"]]
    for g in range(3):
        rows.append(p["grp_proj_W"][g] * p["grp_norm_w"][g][None, :])
        bias.append(p["grp_proj_W"][g] @ p["grp_norm_b"][g]
                    + p["grp_proj_b"][g])
    wsm = jnp.concatenate(rows, 0)
    bsm = jnp.concatenate(bias)[None, :]

    enw = p["exp_norm_w"].reshape(6, 1, d)
    enb = p["exp_norm_b"].reshape(6, d)
    wg6 = (p["exp_gate_W"].reshape(6, DFF, d) * enw).astype(_BF16)
    wv6 = (p["exp_val_W"].reshape(6, DFF, d) * enw).astype(_BF16)
    wo6 = p["exp_out_W"].reshape(6, d, DFF).astype(_BF16)
    bg6 = jnp.einsum("efd,ed->ef", p["exp_gate_W"].reshape(6, DFF, d),
                     enb)[:, None, :].astype(_BF16)
    bv6 = jnp.einsum("efd,ed->ef", p["exp_val_W"].reshape(6, DFF, d),
                     enb)[:, None, :].astype(_BF16)

    wa, ba = p["br_norm_a_w"], p["br_norm_a_b"]
    wb, bb = p["br_norm_b_w"], p["br_norm_b_b"]
    wqf = (p["br_Wq_f"] * wa[None, :]).astype(_BF16)
    wkb = (p["br_Wk_b"] * wa[None, :]).astype(_BF16)
    wvb = (p["br_Wv_b"] * wa[None, :]).astype(_BF16)
    wqb = (p["br_Wq_b"] * wb[None, :]).astype(_BF16)
    wkf = (p["br_Wk_f"] * wb[None, :]).astype(_BF16)
    wvf = (p["br_Wv_f"] * wb[None, :]).astype(_BF16)
    qkv_bias = jnp.stack([p["br_Wq_f"] @ ba, p["br_Wk_f"] @ bb,
                          p["br_Wv_f"] @ bb, p["br_Wq_b"] @ bb,
                          p["br_Wk_b"] @ ba, p["br_Wv_b"] @ ba])

    wof = p["br_Wo_f"].astype(_BF16)
    wob = p["br_Wo_b"].astype(_BF16)
    wgt = (p["br_ffn_gate_W"] * p["br_norm_ffn_w"][None, :]).astype(_BF16)
    wvt = (p["br_ffn_val_W"] * p["br_norm_ffn_w"][None, :]).astype(_BF16)
    bgf = (p["br_ffn_gate_W"] @ p["br_norm_ffn_b"])[None, :].astype(_BF16)
    bvf = (p["br_ffn_val_W"] @ p["br_norm_ffn_b"])[None, :].astype(_BF16)
    wout = p["br_ffn_out_W"].astype(_BF16)

    temp = jnp.clip(jnp.exp(p["log_temp"]), 0.1, 5.0)
    alpha = jax.nn.sigmoid(p["br_log_alpha"])
    gsig = jax.nn.sigmoid(p["br_gate"])
    scal = jnp.zeros((1, 128), _F32)
    scal = scal.at[0, 0].set(temp).at[0, 1].set(alpha).at[0, 2].set(gsig)

    nb = n // _BN

    xhat, cw, lsum = pl.pallas_call(
        _router_kernel,
        grid=(n // _BA,),
        in_specs=[
            pl.BlockSpec((_BA, d), lambda i: (i, 0)),
            pl.BlockSpec((12, d), lambda i: (0, 0)),
            pl.BlockSpec((1, 12), lambda i: (0, 0)),
            pl.BlockSpec((1, 128), lambda i: (0, 0)),
            pl.BlockSpec((64, 6), lambda i: (0, 0)),
            pl.BlockSpec((3, 6), lambda i: (0, 0)),
        ],
        out_specs=[
            pl.BlockSpec((_BA, d), lambda i: (i, 0)),
            pl.BlockSpec((6, _BA, 8), lambda i: (0, i, 0)),
            pl.BlockSpec((1, 128), lambda i: (0, 0)),
        ],
        out_shape=[
            jax.ShapeDtypeStruct((n, d), _BF16),
            jax.ShapeDtypeStruct((6, n, 8), _F32),
            jax.ShapeDtypeStruct((1, 128), _F32),
        ],
        compiler_params=pltpu.CompilerParams(
            dimension_semantics=("arbitrary",)),
    )(xf, wsm, bsm, scal, jnp.asarray(_HEXG_NP), jnp.asarray(_ANCH_NP))

    moe, xab = pl.pallas_call(
        _expert_kernel,
        grid=(nb, 6),
        in_specs=[
            pl.BlockSpec((_BB, d), lambda i, j: (i, 0)),
            pl.BlockSpec((1, DFF, d), lambda i, j: (j, 0, 0)),
            pl.BlockSpec((1, DFF, d), lambda i, j: (j, 0, 0)),
            pl.BlockSpec((1, d, DFF), lambda i, j: (j, 0, 0)),
            pl.BlockSpec((1, 1, DFF), lambda i, j: (j, 0, 0)),
            pl.BlockSpec((1, 1, DFF), lambda i, j: (j, 0, 0)),
            pl.BlockSpec((1, _BB, 8), lambda i, j: (j, i, 0)),
        ],
        out_specs=[
            pl.BlockSpec((_BB, d), lambda i, j: (i, 0)),
            pl.BlockSpec((2, _BB, d), lambda i, j: (0, i, 0)),
        ],
        out_shape=[
            jax.ShapeDtypeStruct((n, d), _BF16),
            jax.ShapeDtypeStruct((2, n, d), _BF16),
        ],
        compiler_params=pltpu.CompilerParams(
            dimension_semantics=("parallel", "arbitrary")),
    )(xhat, wg6, wv6, wo6, bg6, bv6, cw)

    q, k, v = pl.pallas_call(
        _qkv_kernel,
        grid=(nb,),
        in_specs=[
            pl.BlockSpec((2, _BN, d), lambda i: (0, i, 0)),
            pl.BlockSpec((d, d), lambda i: (0, 0)),
            pl.BlockSpec((d, d), lambda i: (0, 0)),
            pl.BlockSpec((d, d), lambda i: (0, 0)),
            pl.BlockSpec((d, d), lambda i: (0, 0)),
            pl.BlockSpec((d, d), lambda i: (0, 0)),
            pl.BlockSpec((d, d), lambda i: (0, 0)),
            pl.BlockSpec((6, d), lambda i: (0, 0)),
        ],
        out_specs=[
            pl.BlockSpec((2, _BN, d), lambda i: (0, i, 0)),
            pl.BlockSpec((2, _BN, d), lambda i: (0, i, 0)),
            pl.BlockSpec((2, _BN, d), lambda i: (0, i, 0)),
        ],
        out_shape=[
            jax.ShapeDtypeStruct((2, n, d), _BF16),
            jax.ShapeDtypeStruct((2, n, d), _BF16),
            jax.ShapeDtypeStruct((2, n, d), _BF16),
        ],
        compiler_params=pltpu.CompilerParams(
            dimension_semantics=("parallel",)),
    )(xab, wqf, wkf, wvf, wqb, wkb, wvb, qkv_bias)

    nh, hd = 4, 256
    nq = t // _BQ
    q4 = q.reshape(2, b, t, d)
    k4 = k.reshape(2, b, t, d)
    v4 = v.reshape(2, b, t, d)
    o_parts = []
    for qi in range(nq):
        kw = (qi + 1) * _BQ
        o_qi = pl.pallas_call(
            functools.partial(_attn_kernel, qi=qi, kw=kw),
            grid=(2 * b * nh,),
            in_specs=[
                pl.BlockSpec((1, 1, _BQ, hd),
                             lambda g: (g // (b * nh), (g // nh) % b,
                                        qi, g % nh)),
                pl.BlockSpec((1, 1, kw, hd),
                             lambda g: (g // (b * nh), (g // nh) % b,
                                        0, g % nh)),
                pl.BlockSpec((1, 1, kw, hd),
                             lambda g: (g // (b * nh), (g // nh) % b,
                                        0, g % nh)),
            ],
            out_specs=pl.BlockSpec((1, 1, _BQ, hd),
                                   lambda g: (g // (b * nh), (g // nh) % b,
                                              0, g % nh)),
            out_shape=jax.ShapeDtypeStruct((2, b, _BQ, d), _BF16),
            compiler_params=pltpu.CompilerParams(
                dimension_semantics=("parallel",)),
        )(q4, k4, v4)
        o_parts.append(o_qi)
    o = jnp.concatenate(o_parts, axis=2).reshape(2, n, d)

    out = pl.pallas_call(
        _final_kernel,
        grid=(nb,),
        in_specs=[
            pl.BlockSpec((_BN, d), lambda i: (i, 0)),
            pl.BlockSpec((_BN, d), lambda i: (i, 0)),
            pl.BlockSpec((2, _BN, d), lambda i: (0, i, 0)),
            pl.BlockSpec((2, _BN, d), lambda i: (0, i, 0)),
            pl.BlockSpec((d, d), lambda i: (0, 0)),
            pl.BlockSpec((d, d), lambda i: (0, 0)),
            pl.BlockSpec((1, 128), lambda i: (0, 0)),
            pl.BlockSpec((DFF, d), lambda i: (0, 0)),
            pl.BlockSpec((1, DFF), lambda i: (0, 0)),
            pl.BlockSpec((DFF, d), lambda i: (0, 0)),
            pl.BlockSpec((1, DFF), lambda i: (0, 0)),
            pl.BlockSpec((d, DFF), lambda i: (0, 0)),
        ],
        out_specs=pl.BlockSpec((_BN, d), lambda i: (i, 0)),
        out_shape=jax.ShapeDtypeStruct((n, d), _F32),
        compiler_params=pltpu.CompilerParams(
            dimension_semantics=("parallel",)),
    )(xf, moe, xab, o, wof, wob, scal, wgt, bgf, wvt, bvf, wout)

    inv_n = 1.0 / n
    mw = lsum[0, 0:3] * inv_n
    total = jnp.sum(mw * jnp.log(mw + 1e-8))
    for g in range(3):
        gmw = lsum[0, 3 + 2 * g:5 + 2 * g] * inv_n
        total = total + jnp.sum(gmw * jnp.log(gmw + 1e-8))
    loss = (0.01 * total).astype(_F32)

    return out.reshape(b, t, d), loss


# speedup vs baseline: 1.3553x; 1.3553x over previous
import functools
import itertools

import numpy as np
import jax
import jax.numpy as jnp
from jax.experimental import pallas as pl
from jax.experimental.pallas import tpu as pltpu

D = 1024
DFF = 2048
_HEXG_NP = np.array(list(itertools.product([-1.0, 1.0], repeat=6)),
                    dtype=np.float32)
_GIDX = [[63, 62], [19, 21], [0, 8]]
_ANCH_NP = np.stack([_HEXG_NP[np.array(i)].mean(0) for i in _GIDX])
_ANCH_NP = (_ANCH_NP /
            np.linalg.norm(_ANCH_NP, axis=-1, keepdims=True)).astype(np.float32)

_BN = 512
_BA = 1024
_BB = 512
_BQ = 1024

_F32 = jnp.float32
_BF16 = jnp.bfloat16
_HI = jax.lax.Precision.HIGHEST


def _std(x):
    m = jnp.mean(x, axis=-1, keepdims=True)
    xc = x - m
    v = jnp.mean(xc * xc, axis=-1, keepdims=True)
    return xc * jax.lax.rsqrt(v + 1e-5)


def _dott(a, w):
    return jax.lax.dot_general(a, w, (((1,), (1,)), ((), ())),
                               preferred_element_type=_F32)


def _dotb(a, w):
    return jax.lax.dot_general(a, w, (((1,), (1,)), ((), ())),
                               preferred_element_type=_BF16)


def _place(val, k, width):
    lane = jax.lax.broadcasted_iota(jnp.int32, (1, width), 1)
    return jnp.where(lane == k, val, 0.0)


def _router_kernel(x_ref, wsm_ref, bsm_ref, sc_ref, hexg_ref, anch_ref,
                   xhat_ref, cw_ref, ls_ref):
    i = pl.program_id(0)
    x = x_ref[...]
    xh = _std(x)
    xhat_ref[...] = xh.astype(_BF16)
    logits = jax.lax.dot_general(
        xh, wsm_ref[...], (((1,), (1,)), ((), ())),
        preferred_element_type=_F32, precision=_HI) + bsm_ref[...]
    hexg = hexg_ref[...]
    anch = anch_ref[...]
    sb = jnp.tanh(logits[:, 0:6])
    temp = sc_ref[0:1, 0:1]
    sim = jax.lax.dot_general(sb, hexg, (((1,), (1,)), ((), ())),
                              preferred_element_type=_F32,
                              precision=_HI) / temp
    sim = sim - jnp.max(sim, axis=1, keepdims=True)
    se = jnp.exp(sim)
    hw = se / jnp.sum(se, axis=1, keepdims=True)
    shex = jnp.dot(hw, hexg, preferred_element_type=_F32,
                   precision=_HI)
    gs = jax.lax.dot_general(shex, anch, (((1,), (1,)), ((), ())),
                             preferred_element_type=_F32,
                             precision=_HI)
    gs = gs - jnp.max(gs, axis=1, keepdims=True)
    ge = jnp.exp(gs)
    gw = ge / jnp.sum(ge, axis=1, keepdims=True)

    row = (_place(jnp.sum(gw[:, 0:1]), 0, 128)
           + _place(jnp.sum(gw[:, 1:2]), 1, 128)
           + _place(jnp.sum(gw[:, 2:3]), 2, 128))
    for g in range(3):
        pair = logits[:, 6 + 2 * g:8 + 2 * g]
        mx = jnp.max(pair, axis=1, keepdims=True)
        pe = jnp.exp(pair - mx)
        ps = jnp.sum(pe, axis=1, keepdims=True)
        w = pe / ps
        sp = w / (jnp.sum(w, axis=1, keepdims=True) + 1e-8)
        row = (row + _place(jnp.sum(w[:, 0:1]), 3 + 2 * g, 128)
               + _place(jnp.sum(w[:, 1:2]), 4 + 2 * g, 128))
        for e in range(2):
            j = 2 * g + e
            cm = gw[:, g:g + 1] * sp[:, e:e + 1]
            cj = _place(cm, 0, 8)
            if g == 0:
                cj = cj + _place(sp[:, e:e + 1], 1, 8)
            if g == 2:
                cj = cj + _place(sp[:, e:e + 1], 2, 8)
            cw_ref[j] = cj

    @pl.when(i == 0)
    def _():
        ls_ref[...] = jnp.zeros_like(ls_ref)

    ls_ref[...] += row


def _expert_kernel(xh_ref, wg_ref, wv_ref, wo_ref, bg_ref, bv_ref, cw_ref,
                   moe_ref, xab_ref):
    j = pl.program_id(1)
    xh = xh_ref[...]
    gate = _dott(xh, wg_ref[0]) + bg_ref[0]
    val = _dott(xh, wv_ref[0]) + bv_ref[0]
    h = (gate * jax.nn.sigmoid(gate) * val).astype(_BF16)
    eo = _dott(h, wo_ref[0])
    c = cw_ref[0]

    @pl.when(j == 0)
    def _():
        moe_ref[...] = c[:, 0:1] * eo
        xab_ref[0] = c[:, 1:2] * eo

    @pl.when(j > 0)
    def _():
        moe_ref[...] += c[:, 0:1] * eo

    @pl.when(j == 1)
    def _():
        xab_ref[0] += c[:, 1:2] * eo

    @pl.when(j == 4)
    def _():
        xab_ref[1] = c[:, 2:3] * eo

    @pl.when(j == 5)
    def _():
        xab_ref[1] += c[:, 2:3] * eo


def _qkv_kernel(xab_ref, wqf_ref, wkf_ref, wvf_ref, wqb_ref, wkb_ref,
                wvb_ref, bias_ref, q_ref, k_ref, v_ref):
    na = _std(xab_ref[0]).astype(_BF16)
    nb_ = _std(xab_ref[1]).astype(_BF16)
    q_ref[0] = (_dott(na, wqf_ref[...]) + bias_ref[0:1]).astype(_BF16)
    k_ref[0] = (_dott(nb_, wkf_ref[...]) + bias_ref[1:2]).astype(_BF16)
    v_ref[0] = (_dott(nb_, wvf_ref[...]) + bias_ref[2:3]).astype(_BF16)
    q_ref[1] = (_dott(nb_, wqb_ref[...]) + bias_ref[3:4]).astype(_BF16)
    k_ref[1] = (_dott(na, wkb_ref[...]) + bias_ref[4:5]).astype(_BF16)
    v_ref[1] = (_dott(na, wvb_ref[...]) + bias_ref[5:6]).astype(_BF16)


def _attn_kernel(q_ref, k_ref, v_ref, o_ref, *, qi, kw):
    q = q_ref[0, 0]
    k = k_ref[0, 0]
    v = v_ref[0, 0]
    s = jax.lax.dot_general(q, k, (((1,), (1,)), ((), ())),
                            preferred_element_type=_F32) * (1.0 / 16.0)
    qpos = qi * _BQ + jax.lax.broadcasted_iota(jnp.int32, (_BQ, kw), 0)
    kpos = jax.lax.broadcasted_iota(jnp.int32, (_BQ, kw), 1)
    s = jnp.where(kpos > qpos, -1e30, s)
    mx = jnp.max(s, axis=1, keepdims=True)
    p = jnp.exp(s - mx)
    l = jnp.sum(p, axis=1, keepdims=True)
    o = jnp.dot(p.astype(_BF16), v, preferred_element_type=_F32) / l
    o_ref[0, 0] = o.astype(_BF16)


def _final_kernel(x_ref, moe_ref, xab_ref, o_ref, wof_ref, wob_ref,
                  sc_ref, wg_ref, bg_ref, wv_ref, bv_ref, wo2_ref, out_ref):
    alpha = sc_ref[0:1, 1:2]
    gsig = sc_ref[0:1, 2:3]
    fwd = _dott(o_ref[0], wof_ref[...])
    bwd = _dott(o_ref[1], wob_ref[...])
    crossed = (alpha * (xab_ref[0] + fwd)
               + (1.0 - alpha) * (xab_ref[1] + bwd))
    hh = _std(crossed).astype(_BF16)
    g = _dott(hh, wg_ref[...]) + bg_ref[...]
    vv = _dott(hh, wv_ref[...]) + bv_ref[...]
    h = (g * jax.nn.sigmoid(g) * vv).astype(_BF16)
    ffn = _dott(h, wo2_ref[...])
    bout = (crossed + ffn) * gsig
    out_ref[...] = x_ref[...] + moe_ref[...] + 0.5 * bout


def kernel(x, params):
    p = params
    b, t, d = x.shape
    n = b * t
    xf = x.reshape(n, d)

    rows = [p["q6_W"] * p["gr_norm_w"][None, :]]
    bias = [p["q6_W"] @ p["gr_norm_b"]]
    for g in range(3):
        rows.append(p["grp_proj_W"][g] * p["grp_norm_w"][g][None, :])
        bias.append(p["grp_proj_W"][g] @ p["grp_norm_b"][g]
                    + p["grp_proj_b"][g])
    wsm = jnp.concatenate(rows, 0)
    bsm = jnp.concatenate(bias)[None, :]

    enw = p["exp_norm_w"].reshape(6, 1, d)
    enb = p["exp_norm_b"].reshape(6, d)
    wg6 = (p["exp_gate_W"].reshape(6, DFF, d) * enw).astype(_BF16)
    wv6 = (p["exp_val_W"].reshape(6, DFF, d) * enw).astype(_BF16)
    wo6 = p["exp_out_W"].reshape(6, d, DFF).astype(_BF16)
    bg6 = jnp.einsum("efd,ed->ef", p["exp_gate_W"].reshape(6, DFF, d),
                     enb)[:, None, :].astype(_BF16)
    bv6 = jnp.einsum("efd,ed->ef", p["exp_val_W"].reshape(6, DFF, d),
                     enb)[:, None, :].astype(_BF16)

    wa, ba = p["br_norm_a_w"], p["br_norm_a_b"]
    wb, bb = p["br_norm_b_w"], p["br_norm_b_b"]
    wqf = (p["br_Wq_f"] * wa[None, :]).astype(_BF16)
    wkb = (p["br_Wk_b"] * wa[None, :]).astype(_BF16)
    wvb = (p["br_Wv_b"] * wa[None, :]).astype(_BF16)
    wqb = (p["br_Wq_b"] * wb[None, :]).astype(_BF16)
    wkf = (p["br_Wk_f"] * wb[None, :]).astype(_BF16)
    wvf = (p["br_Wv_f"] * wb[None, :]).astype(_BF16)
    qkv_bias = jnp.stack([p["br_Wq_f"] @ ba, p["br_Wk_f"] @ bb,
                          p["br_Wv_f"] @ bb, p["br_Wq_b"] @ bb,
                          p["br_Wk_b"] @ ba, p["br_Wv_b"] @ ba])

    wof = p["br_Wo_f"].astype(_BF16)
    wob = p["br_Wo_b"].astype(_BF16)
    wgt = (p["br_ffn_gate_W"] * p["br_norm_ffn_w"][None, :]).astype(_BF16)
    wvt = (p["br_ffn_val_W"] * p["br_norm_ffn_w"][None, :]).astype(_BF16)
    bgf = (p["br_ffn_gate_W"] @ p["br_norm_ffn_b"])[None, :].astype(_BF16)
    bvf = (p["br_ffn_val_W"] @ p["br_norm_ffn_b"])[None, :].astype(_BF16)
    wout = p["br_ffn_out_W"].astype(_BF16)

    temp = jnp.clip(jnp.exp(p["log_temp"]), 0.1, 5.0)
    alpha = jax.nn.sigmoid(p["br_log_alpha"])
    gsig = jax.nn.sigmoid(p["br_gate"])
    scal = jnp.zeros((1, 128), _F32)
    scal = scal.at[0, 0].set(temp).at[0, 1].set(alpha).at[0, 2].set(gsig)

    nb = n // _BN

    xhat, cw, lsum = pl.pallas_call(
        _router_kernel,
        grid=(n // _BA,),
        in_specs=[
            pl.BlockSpec((_BA, d), lambda i: (i, 0)),
            pl.BlockSpec((12, d), lambda i: (0, 0)),
            pl.BlockSpec((1, 12), lambda i: (0, 0)),
            pl.BlockSpec((1, 128), lambda i: (0, 0)),
            pl.BlockSpec((64, 6), lambda i: (0, 0)),
            pl.BlockSpec((3, 6), lambda i: (0, 0)),
        ],
        out_specs=[
            pl.BlockSpec((_BA, d), lambda i: (i, 0)),
            pl.BlockSpec((6, _BA, 8), lambda i: (0, i, 0)),
            pl.BlockSpec((1, 128), lambda i: (0, 0)),
        ],
        out_shape=[
            jax.ShapeDtypeStruct((n, d), _BF16),
            jax.ShapeDtypeStruct((6, n, 8), _F32),
            jax.ShapeDtypeStruct((1, 128), _F32),
        ],
        compiler_params=pltpu.CompilerParams(
            dimension_semantics=("arbitrary",)),
    )(xf, wsm, bsm, scal, jnp.asarray(_HEXG_NP), jnp.asarray(_ANCH_NP))

    moe, xab = pl.pallas_call(
        _expert_kernel,
        grid=(nb, 6),
        in_specs=[
            pl.BlockSpec((_BB, d), lambda i, j: (i, 0)),
            pl.BlockSpec((1, DFF, d), lambda i, j: (j, 0, 0)),
            pl.BlockSpec((1, DFF, d), lambda i, j: (j, 0, 0)),
            pl.BlockSpec((1, d, DFF), lambda i, j: (j, 0, 0)),
            pl.BlockSpec((1, 1, DFF), lambda i, j: (j, 0, 0)),
            pl.BlockSpec((1, 1, DFF), lambda i, j: (j, 0, 0)),
            pl.BlockSpec((1, _BB, 8), lambda i, j: (j, i, 0)),
        ],
        out_specs=[
            pl.BlockSpec((_BB, d), lambda i, j: (i, 0)),
            pl.BlockSpec((2, _BB, d), lambda i, j: (0, i, 0)),
        ],
        out_shape=[
            jax.ShapeDtypeStruct((n, d), _F32),
            jax.ShapeDtypeStruct((2, n, d), _F32),
        ],
        compiler_params=pltpu.CompilerParams(
            dimension_semantics=("parallel", "arbitrary")),
    )(xhat, wg6, wv6, wo6, bg6, bv6, cw)

    q, k, v = pl.pallas_call(
        _qkv_kernel,
        grid=(nb,),
        in_specs=[
            pl.BlockSpec((2, _BN, d), lambda i: (0, i, 0)),
            pl.BlockSpec((d, d), lambda i: (0, 0)),
            pl.BlockSpec((d, d), lambda i: (0, 0)),
            pl.BlockSpec((d, d), lambda i: (0, 0)),
            pl.BlockSpec((d, d), lambda i: (0, 0)),
            pl.BlockSpec((d, d), lambda i: (0, 0)),
            pl.BlockSpec((d, d), lambda i: (0, 0)),
            pl.BlockSpec((6, d), lambda i: (0, 0)),
        ],
        out_specs=[
            pl.BlockSpec((2, _BN, d), lambda i: (0, i, 0)),
            pl.BlockSpec((2, _BN, d), lambda i: (0, i, 0)),
            pl.BlockSpec((2, _BN, d), lambda i: (0, i, 0)),
        ],
        out_shape=[
            jax.ShapeDtypeStruct((2, n, d), _BF16),
            jax.ShapeDtypeStruct((2, n, d), _BF16),
            jax.ShapeDtypeStruct((2, n, d), _BF16),
        ],
        compiler_params=pltpu.CompilerParams(
            dimension_semantics=("parallel",)),
    )(xab, wqf, wkf, wvf, wqb, wkb, wvb, qkv_bias)

    nh, hd = 4, 256
    nq = t // _BQ
    q4 = q.reshape(2, b, t, d)
    k4 = k.reshape(2, b, t, d)
    v4 = v.reshape(2, b, t, d)
    o_parts = []
    for qi in range(nq):
        kw = (qi + 1) * _BQ
        o_qi = pl.pallas_call(
            functools.partial(_attn_kernel, qi=qi, kw=kw),
            grid=(2 * b * nh,),
            in_specs=[
                pl.BlockSpec((1, 1, _BQ, hd),
                             lambda g: (g // (b * nh), (g // nh) % b,
                                        qi, g % nh)),
                pl.BlockSpec((1, 1, kw, hd),
                             lambda g: (g // (b * nh), (g // nh) % b,
                                        0, g % nh)),
                pl.BlockSpec((1, 1, kw, hd),
                             lambda g: (g // (b * nh), (g // nh) % b,
                                        0, g % nh)),
            ],
            out_specs=pl.BlockSpec((1, 1, _BQ, hd),
                                   lambda g: (g // (b * nh), (g // nh) % b,
                                              0, g % nh)),
            out_shape=jax.ShapeDtypeStruct((2, b, _BQ, d), _BF16),
            compiler_params=pltpu.CompilerParams(
                dimension_semantics=("parallel",)),
        )(q4, k4, v4)
        o_parts.append(o_qi)
    o = jnp.concatenate(o_parts, axis=2).reshape(2, n, d)

    out = pl.pallas_call(
        _final_kernel,
        grid=(nb,),
        in_specs=[
            pl.BlockSpec((_BN, d), lambda i: (i, 0)),
            pl.BlockSpec((_BN, d), lambda i: (i, 0)),
            pl.BlockSpec((2, _BN, d), lambda i: (0, i, 0)),
            pl.BlockSpec((2, _BN, d), lambda i: (0, i, 0)),
            pl.BlockSpec((d, d), lambda i: (0, 0)),
            pl.BlockSpec((d, d), lambda i: (0, 0)),
            pl.BlockSpec((1, 128), lambda i: (0, 0)),
            pl.BlockSpec((DFF, d), lambda i: (0, 0)),
            pl.BlockSpec((1, DFF), lambda i: (0, 0)),
            pl.BlockSpec((DFF, d), lambda i: (0, 0)),
            pl.BlockSpec((1, DFF), lambda i: (0, 0)),
            pl.BlockSpec((d, DFF), lambda i: (0, 0)),
        ],
        out_specs=pl.BlockSpec((_BN, d), lambda i: (i, 0)),
        out_shape=jax.ShapeDtypeStruct((n, d), _F32),
        compiler_params=pltpu.CompilerParams(
            dimension_semantics=("parallel",)),
    )(xf, moe, xab, o, wof, wob, scal, wgt, bgf, wvt, bvf, wout)

    inv_n = 1.0 / n
    mw = lsum[0, 0:3] * inv_n
    total = jnp.sum(mw * jnp.log(mw + 1e-8))
    for g in range(3):
        gmw = lsum[0, 3 + 2 * g:5 + 2 * g] * inv_n
        total = total + jnp.sum(gmw * jnp.log(gmw + 1e-8))
    loss = (0.01 * total).astype(_F32)

    return out.reshape(b, t, d), loss


# speedup vs baseline: 1.3590x; 1.0027x over previous
import functools
import itertools

import numpy as np
import jax
import jax.numpy as jnp
from jax.experimental import pallas as pl
from jax.experimental.pallas import tpu as pltpu

D = 1024
DFF = 2048
_HEXG_NP = np.array(list(itertools.product([-1.0, 1.0], repeat=6)),
                    dtype=np.float32)
_GIDX = [[63, 62], [19, 21], [0, 8]]
_ANCH_NP = np.stack([_HEXG_NP[np.array(i)].mean(0) for i in _GIDX])
_ANCH_NP = (_ANCH_NP /
            np.linalg.norm(_ANCH_NP, axis=-1, keepdims=True)).astype(np.float32)

_BN = 512
_BA = 1024
_BB = 512
_BQ = 1024

_F32 = jnp.float32
_BF16 = jnp.bfloat16
_HI = jax.lax.Precision.HIGHEST


def _std(x):
    m = jnp.mean(x, axis=-1, keepdims=True)
    xc = x - m
    v = jnp.mean(xc * xc, axis=-1, keepdims=True)
    return xc * jax.lax.rsqrt(v + 1e-5)


def _dott(a, w):
    return jax.lax.dot_general(a, w, (((1,), (1,)), ((), ())),
                               preferred_element_type=_F32)


def _dotb(a, w):
    return jax.lax.dot_general(a, w, (((1,), (1,)), ((), ())),
                               preferred_element_type=_BF16)


def _place(val, k, width):
    lane = jax.lax.broadcasted_iota(jnp.int32, (1, width), 1)
    return jnp.where(lane == k, val, 0.0)


def _router_kernel(x_ref, wsm_ref, bsm_ref, sc_ref, hexg_ref, anch_ref,
                   xhat_ref, cw_ref, ls_ref):
    i = pl.program_id(0)
    x = x_ref[...]
    xh = _std(x)
    xhat_ref[...] = xh.astype(_BF16)
    logits = jax.lax.dot_general(
        xh, wsm_ref[...], (((1,), (1,)), ((), ())),
        preferred_element_type=_F32, precision=_HI) + bsm_ref[...]
    hexg = hexg_ref[...]
    anch = anch_ref[...]
    sb = jnp.tanh(logits[:, 0:6])
    temp = sc_ref[0:1, 0:1]
    sim = jax.lax.dot_general(sb, hexg, (((1,), (1,)), ((), ())),
                              preferred_element_type=_F32,
                              precision=_HI) / temp
    sim = sim - jnp.max(sim, axis=1, keepdims=True)
    se = jnp.exp(sim)
    hw = se / jnp.sum(se, axis=1, keepdims=True)
    shex = jnp.dot(hw, hexg, preferred_element_type=_F32,
                   precision=_HI)
    gs = jax.lax.dot_general(shex, anch, (((1,), (1,)), ((), ())),
                             preferred_element_type=_F32,
                             precision=_HI)
    gs = gs - jnp.max(gs, axis=1, keepdims=True)
    ge = jnp.exp(gs)
    gw = ge / jnp.sum(ge, axis=1, keepdims=True)

    row = (_place(jnp.sum(gw[:, 0:1]), 0, 128)
           + _place(jnp.sum(gw[:, 1:2]), 1, 128)
           + _place(jnp.sum(gw[:, 2:3]), 2, 128))
    for g in range(3):
        pair = logits[:, 6 + 2 * g:8 + 2 * g]
        mx = jnp.max(pair, axis=1, keepdims=True)
        pe = jnp.exp(pair - mx)
        ps = jnp.sum(pe, axis=1, keepdims=True)
        w = pe / ps
        sp = w / (jnp.sum(w, axis=1, keepdims=True) + 1e-8)
        row = (row + _place(jnp.sum(w[:, 0:1]), 3 + 2 * g, 128)
               + _place(jnp.sum(w[:, 1:2]), 4 + 2 * g, 128))
        for e in range(2):
            j = 2 * g + e
            cm = gw[:, g:g + 1] * sp[:, e:e + 1]
            cj = _place(cm, 0, 8)
            if g == 0:
                cj = cj + _place(sp[:, e:e + 1], 1, 8)
            if g == 2:
                cj = cj + _place(sp[:, e:e + 1], 2, 8)
            cw_ref[j] = cj

    @pl.when(i == 0)
    def _():
        ls_ref[...] = jnp.zeros_like(ls_ref)

    ls_ref[...] += row


def _expert_kernel(xh_ref, wg_ref, wv_ref, wo_ref, bg_ref, bv_ref, cw_ref,
                   moe_ref, xab_ref):
    j = pl.program_id(1)
    xh = xh_ref[...]
    gate = _dott(xh, wg_ref[0]) + bg_ref[0]
    val = _dott(xh, wv_ref[0]) + bv_ref[0]
    h = (gate * jax.nn.sigmoid(gate) * val).astype(_BF16)
    eo = _dott(h, wo_ref[0])
    c = cw_ref[0]

    @pl.when(j == 0)
    def _():
        moe_ref[...] = c[:, 0:1] * eo
        xab_ref[0] = c[:, 1:2] * eo

    @pl.when(j > 0)
    def _():
        moe_ref[...] += c[:, 0:1] * eo

    @pl.when(j == 1)
    def _():
        xab_ref[0] += c[:, 1:2] * eo

    @pl.when(j == 4)
    def _():
        xab_ref[1] = c[:, 2:3] * eo

    @pl.when(j == 5)
    def _():
        xab_ref[1] += c[:, 2:3] * eo


def _qkv_kernel(xab_ref, wqf_ref, wkf_ref, wvf_ref, wqb_ref, wkb_ref,
                wvb_ref, bias_ref, q_ref, k_ref, v_ref):
    na = _std(xab_ref[0]).astype(_BF16)
    nb_ = _std(xab_ref[1]).astype(_BF16)
    q_ref[0] = (_dott(na, wqf_ref[...]) + bias_ref[0:1]).astype(_BF16)
    k_ref[0] = (_dott(nb_, wkf_ref[...]) + bias_ref[1:2]).astype(_BF16)
    v_ref[0] = (_dott(nb_, wvf_ref[...]) + bias_ref[2:3]).astype(_BF16)
    q_ref[1] = (_dott(nb_, wqb_ref[...]) + bias_ref[3:4]).astype(_BF16)
    k_ref[1] = (_dott(na, wkb_ref[...]) + bias_ref[4:5]).astype(_BF16)
    v_ref[1] = (_dott(na, wvb_ref[...]) + bias_ref[5:6]).astype(_BF16)


def _attn_kernel(q_ref, k_ref, v_ref, o_ref, *, qi, kw):
    q = q_ref[0, 0]
    k = k_ref[0, 0]
    v = v_ref[0, 0]
    bw = qi * _BQ
    s = jax.lax.dot_general(q, k, (((1,), (1,)), ((), ())),
                            preferred_element_type=_F32)
    rr = jax.lax.broadcasted_iota(jnp.int32, (_BQ, _BQ), 0)
    cc = jax.lax.broadcasted_iota(jnp.int32, (_BQ, _BQ), 1)
    sd = jnp.where(cc > rr, -1e30, s[:, bw:])
    mx = jnp.max(sd, axis=1, keepdims=True)
    if qi > 0:
        sb = s[:, :bw]
        mx = jnp.maximum(mx, jnp.max(sb, axis=1, keepdims=True))
    pd = jnp.exp(sd - mx)
    l = jnp.sum(pd, axis=1, keepdims=True)
    acc = jnp.dot(pd.astype(_BF16), v[bw:],
                  preferred_element_type=_F32)
    if qi > 0:
        pb = jnp.exp(sb - mx)
        l = l + jnp.sum(pb, axis=1, keepdims=True)
        acc = acc + jnp.dot(pb.astype(_BF16), v[:bw],
                            preferred_element_type=_F32)
    o_ref[0, 0] = (acc / l).astype(_BF16)


def _final_kernel(x_ref, moe_ref, xab_ref, o_ref, wof_ref, wob_ref,
                  sc_ref, wg_ref, bg_ref, wv_ref, bv_ref, wo2_ref, out_ref):
    alpha = sc_ref[0:1, 1:2]
    gsig = sc_ref[0:1, 2:3]
    fwd = _dott(o_ref[0], wof_ref[...])
    bwd = _dott(o_ref[1], wob_ref[...])
    crossed = (alpha * (xab_ref[0] + fwd)
               + (1.0 - alpha) * (xab_ref[1] + bwd))
    hh = _std(crossed).astype(_BF16)
    g = _dott(hh, wg_ref[...]) + bg_ref[...]
    vv = _dott(hh, wv_ref[...]) + bv_ref[...]
    h = (g * jax.nn.sigmoid(g) * vv).astype(_BF16)
    ffn = _dott(h, wo2_ref[...])
    bout = (crossed + ffn) * gsig
    out_ref[...] = x_ref[...] + moe_ref[...] + 0.5 * bout


def kernel(x, params):
    p = params
    b, t, d = x.shape
    n = b * t
    xf = x.reshape(n, d)

    rows = [p["q6_W"] * p["gr_norm_w"][None, :]]
    bias = [p["q6_W"] @ p["gr_norm_b"]]
    for g in range(3):
        rows.append(p["grp_proj_W"][g] * p["grp_norm_w"][g][None, :])
        bias.append(p["grp_proj_W"][g] @ p["grp_norm_b"][g]
                    + p["grp_proj_b"][g])
    wsm = jnp.concatenate(rows, 0)
    bsm = jnp.concatenate(bias)[None, :]

    enw = p["exp_norm_w"].reshape(6, 1, d)
    enb = p["exp_norm_b"].reshape(6, d)
    wg6 = (p["exp_gate_W"].reshape(6, DFF, d) * enw).astype(_BF16)
    wv6 = (p["exp_val_W"].reshape(6, DFF, d) * enw).astype(_BF16)
    wo6 = p["exp_out_W"].reshape(6, d, DFF).astype(_BF16)
    bg6 = jnp.einsum("efd,ed->ef", p["exp_gate_W"].reshape(6, DFF, d),
                     enb)[:, None, :].astype(_BF16)
    bv6 = jnp.einsum("efd,ed->ef", p["exp_val_W"].reshape(6, DFF, d),
                     enb)[:, None, :].astype(_BF16)

    wa, ba = p["br_norm_a_w"], p["br_norm_a_b"]
    wb, bb = p["br_norm_b_w"], p["br_norm_b_b"]
    wqf = (p["br_Wq_f"] * (wa[None, :] / 16.0)).astype(_BF16)
    wkb = (p["br_Wk_b"] * wa[None, :]).astype(_BF16)
    wvb = (p["br_Wv_b"] * wa[None, :]).astype(_BF16)
    wqb = (p["br_Wq_b"] * (wb[None, :] / 16.0)).astype(_BF16)
    wkf = (p["br_Wk_f"] * wb[None, :]).astype(_BF16)
    wvf = (p["br_Wv_f"] * wb[None, :]).astype(_BF16)
    qkv_bias = jnp.stack([(p["br_Wq_f"] @ ba) / 16.0, p["br_Wk_f"] @ bb,
                          p["br_Wv_f"] @ bb, (p["br_Wq_b"] @ bb) / 16.0,
                          p["br_Wk_b"] @ ba, p["br_Wv_b"] @ ba])

    wof = p["br_Wo_f"].astype(_BF16)
    wob = p["br_Wo_b"].astype(_BF16)
    wgt = (p["br_ffn_gate_W"] * p["br_norm_ffn_w"][None, :]).astype(_BF16)
    wvt = (p["br_ffn_val_W"] * p["br_norm_ffn_w"][None, :]).astype(_BF16)
    bgf = (p["br_ffn_gate_W"] @ p["br_norm_ffn_b"])[None, :].astype(_BF16)
    bvf = (p["br_ffn_val_W"] @ p["br_norm_ffn_b"])[None, :].astype(_BF16)
    wout = p["br_ffn_out_W"].astype(_BF16)

    temp = jnp.clip(jnp.exp(p["log_temp"]), 0.1, 5.0)
    alpha = jax.nn.sigmoid(p["br_log_alpha"])
    gsig = jax.nn.sigmoid(p["br_gate"])
    scal = jnp.zeros((1, 128), _F32)
    scal = scal.at[0, 0].set(temp).at[0, 1].set(alpha).at[0, 2].set(gsig)

    nb = n // _BN

    xhat, cw, lsum = pl.pallas_call(
        _router_kernel,
        grid=(n // _BA,),
        in_specs=[
            pl.BlockSpec((_BA, d), lambda i: (i, 0)),
            pl.BlockSpec((12, d), lambda i: (0, 0)),
            pl.BlockSpec((1, 12), lambda i: (0, 0)),
            pl.BlockSpec((1, 128), lambda i: (0, 0)),
            pl.BlockSpec((64, 6), lambda i: (0, 0)),
            pl.BlockSpec((3, 6), lambda i: (0, 0)),
        ],
        out_specs=[
            pl.BlockSpec((_BA, d), lambda i: (i, 0)),
            pl.BlockSpec((6, _BA, 8), lambda i: (0, i, 0)),
            pl.BlockSpec((1, 128), lambda i: (0, 0)),
        ],
        out_shape=[
            jax.ShapeDtypeStruct((n, d), _BF16),
            jax.ShapeDtypeStruct((6, n, 8), _F32),
            jax.ShapeDtypeStruct((1, 128), _F32),
        ],
        compiler_params=pltpu.CompilerParams(
            dimension_semantics=("arbitrary",)),
    )(xf, wsm, bsm, scal, jnp.asarray(_HEXG_NP), jnp.asarray(_ANCH_NP))

    moe, xab = pl.pallas_call(
        _expert_kernel,
        grid=(nb, 6),
        in_specs=[
            pl.BlockSpec((_BB, d), lambda i, j: (i, 0)),
            pl.BlockSpec((1, DFF, d), lambda i, j: (j, 0, 0)),
            pl.BlockSpec((1, DFF, d), lambda i, j: (j, 0, 0)),
            pl.BlockSpec((1, d, DFF), lambda i, j: (j, 0, 0)),
            pl.BlockSpec((1, 1, DFF), lambda i, j: (j, 0, 0)),
            pl.BlockSpec((1, 1, DFF), lambda i, j: (j, 0, 0)),
            pl.BlockSpec((1, _BB, 8), lambda i, j: (j, i, 0)),
        ],
        out_specs=[
            pl.BlockSpec((_BB, d), lambda i, j: (i, 0)),
            pl.BlockSpec((2, _BB, d), lambda i, j: (0, i, 0)),
        ],
        out_shape=[
            jax.ShapeDtypeStruct((n, d), _F32),
            jax.ShapeDtypeStruct((2, n, d), _F32),
        ],
        compiler_params=pltpu.CompilerParams(
            dimension_semantics=("parallel", "arbitrary")),
    )(xhat, wg6, wv6, wo6, bg6, bv6, cw)

    q, k, v = pl.pallas_call(
        _qkv_kernel,
        grid=(nb,),
        in_specs=[
            pl.BlockSpec((2, _BN, d), lambda i: (0, i, 0)),
            pl.BlockSpec((d, d), lambda i: (0, 0)),
            pl.BlockSpec((d, d), lambda i: (0, 0)),
            pl.BlockSpec((d, d), lambda i: (0, 0)),
            pl.BlockSpec((d, d), lambda i: (0, 0)),
            pl.BlockSpec((d, d), lambda i: (0, 0)),
            pl.BlockSpec((d, d), lambda i: (0, 0)),
            pl.BlockSpec((6, d), lambda i: (0, 0)),
        ],
        out_specs=[
            pl.BlockSpec((2, _BN, d), lambda i: (0, i, 0)),
            pl.BlockSpec((2, _BN, d), lambda i: (0, i, 0)),
            pl.BlockSpec((2, _BN, d), lambda i: (0, i, 0)),
        ],
        out_shape=[
            jax.ShapeDtypeStruct((2, n, d), _BF16),
            jax.ShapeDtypeStruct((2, n, d), _BF16),
            jax.ShapeDtypeStruct((2, n, d), _BF16),
        ],
        compiler_params=pltpu.CompilerParams(
            dimension_semantics=("parallel",)),
    )(xab, wqf, wkf, wvf, wqb, wkb, wvb, qkv_bias)

    nh, hd = 4, 256
    nq = t // _BQ
    q4 = q.reshape(2, b, t, d)
    k4 = k.reshape(2, b, t, d)
    v4 = v.reshape(2, b, t, d)
    o_parts = []
    for qi in range(nq):
        kw = (qi + 1) * _BQ
        o_qi = pl.pallas_call(
            functools.partial(_attn_kernel, qi=qi, kw=kw),
            grid=(2 * b * nh,),
            in_specs=[
                pl.BlockSpec((1, 1, _BQ, hd),
                             lambda g: (g // (b * nh), (g // nh) % b,
                                        qi, g % nh)),
                pl.BlockSpec((1, 1, kw, hd),
                             lambda g: (g // (b * nh), (g // nh) % b,
                                        0, g % nh)),
                pl.BlockSpec((1, 1, kw, hd),
                             lambda g: (g // (b * nh), (g // nh) % b,
                                        0, g % nh)),
            ],
            out_specs=pl.BlockSpec((1, 1, _BQ, hd),
                                   lambda g: (g // (b * nh), (g // nh) % b,
                                              0, g % nh)),
            out_shape=jax.ShapeDtypeStruct((2, b, _BQ, d), _BF16),
            compiler_params=pltpu.CompilerParams(
                dimension_semantics=("parallel",)),
        )(q4, k4, v4)
        o_parts.append(o_qi)
    o = jnp.concatenate(o_parts, axis=2).reshape(2, n, d)

    out = pl.pallas_call(
        _final_kernel,
        grid=(nb,),
        in_specs=[
            pl.BlockSpec((_BN, d), lambda i: (i, 0)),
            pl.BlockSpec((_BN, d), lambda i: (i, 0)),
            pl.BlockSpec((2, _BN, d), lambda i: (0, i, 0)),
            pl.BlockSpec((2, _BN, d), lambda i: (0, i, 0)),
            pl.BlockSpec((d, d), lambda i: (0, 0)),
            pl.BlockSpec((d, d), lambda i: (0, 0)),
            pl.BlockSpec((1, 128), lambda i: (0, 0)),
            pl.BlockSpec((DFF, d), lambda i: (0, 0)),
            pl.BlockSpec((1, DFF), lambda i: (0, 0)),
            pl.BlockSpec((DFF, d), lambda i: (0, 0)),
            pl.BlockSpec((1, DFF), lambda i: (0, 0)),
            pl.BlockSpec((d, DFF), lambda i: (0, 0)),
        ],
        out_specs=pl.BlockSpec((_BN, d), lambda i: (i, 0)),
        out_shape=jax.ShapeDtypeStruct((n, d), _F32),
        compiler_params=pltpu.CompilerParams(
            dimension_semantics=("parallel",)),
    )(xf, moe, xab, o, wof, wob, scal, wgt, bgf, wvt, bvf, wout)

    inv_n = 1.0 / n
    mw = lsum[0, 0:3] * inv_n
    total = jnp.sum(mw * jnp.log(mw + 1e-8))
    for g in range(3):
        gmw = lsum[0, 3 + 2 * g:5 + 2 * g] * inv_n
        total = total + jnp.sum(gmw * jnp.log(gmw + 1e-8))
    loss = (0.01 * total).astype(_F32)

    return out.reshape(b, t, d), loss
